# baseline (speedup 1.0000x reference)
import jax
import jax.numpy as jnp
from jax import lax
from jax.experimental import pallas as pl
from jax.experimental.pallas import tpu as pltpu
from jax.experimental.pallas import tpu_sc as plsc

N = 10000
D = 128
H = 8
C = 16
NC = 2
NS = 16
NW = NC * NS
BA = 128
B = 128

NPAD = 10048
RPW = NPAD // NW
RPS = NPAD // NS


def _bcast_lane(row16, lane):
    idx = jnp.full((16, 1), lane, jnp.int32)
    dnums = lax.GatherDimensionNumbers(
        offset_dims=(), collapsed_slice_dims=(0,), start_index_map=(0,))
    return lax.gather(row16, idx, dnums, (1,),
                      mode=lax.GatherScatterMode.PROMISE_IN_BOUNDS)


def _tc_proj_body(x_ref, wt_ref, ss_ref, sd_ref,
                  h_ref, as_ref, ad_ref, mxs_ref, mxd_ref):
    i = pl.program_id(0)
    h = jnp.dot(x_ref[...], wt_ref[...], preferred_element_type=jnp.float32)
    h_ref[...] = h
    a_s = jnp.dot(h, ss_ref[...], preferred_element_type=jnp.float32)
    a_d = jnp.dot(h, sd_ref[...], preferred_element_type=jnp.float32)
    as_ref[...] = a_s
    ad_ref[...] = a_d
    ms = jnp.broadcast_to(jnp.max(a_s, axis=0, keepdims=True), (8, 2 * H))
    md = jnp.broadcast_to(jnp.max(a_d, axis=0, keepdims=True), (8, 2 * H))

    @pl.when(i == 0)
    def _():
        mxs_ref[...] = ms
        mxd_ref[...] = md

    @pl.when(i > 0)
    def _():
        mxs_ref[...] = jnp.maximum(mxs_ref[...], ms)
        mxd_ref[...] = jnp.maximum(mxd_ref[...], md)


def _tc_proj(xp, wt, s_src, s_dst):
    rb = 64
    assert NPAD % rb == 0
    grid = (NPAD // rb,)
    return pl.pallas_call(
        _tc_proj_body,
        grid=grid,
        in_specs=[
            pl.BlockSpec((rb, D), lambda i: (i, 0)),
            pl.BlockSpec((D, D), lambda i: (0, 0)),
            pl.BlockSpec((D, 2 * H), lambda i: (0, 0)),
            pl.BlockSpec((D, 2 * H), lambda i: (0, 0)),
        ],
        out_specs=[
            pl.BlockSpec((rb, D), lambda i: (i, 0)),
            pl.BlockSpec((rb, 2 * H), lambda i: (i, 0)),
            pl.BlockSpec((rb, 2 * H), lambda i: (i, 0)),
            pl.BlockSpec((8, 2 * H), lambda i: (0, 0)),
            pl.BlockSpec((8, 2 * H), lambda i: (0, 0)),
        ],
        out_shape=[
            jax.ShapeDtypeStruct((NPAD, D), jnp.float32),
            jax.ShapeDtypeStruct((NPAD, 2 * H), jnp.float32),
            jax.ShapeDtypeStruct((NPAD, 2 * H), jnp.float32),
            jax.ShapeDtypeStruct((8, 2 * H), jnp.float32),
            jax.ShapeDtypeStruct((8, 2 * H), jnp.float32),
        ],
    )(xp, wt, s_src, s_dst)


def _edge_accum(src, dst, atab_s, atab_d, htab, g16, z128, z16, epad):
    epw = epad // NW
    nblk = epw // BA
    assert nblk % 2 == 0
    mesh = plsc.VectorSubcoreMesh(core_axis_name="c", subcore_axis_name="s",
                                  num_cores=NC, num_subcores=NS)

    def body(src_h, dst_h, as_h, ad_h, h_h, g_h, z128_h, z16_h,
             exp_h, macc_h, sacc_h,
             msg_s, sum_s,
             srcn0, srcn1, dstn0, dstn1,
             asv1, adv1, hv0, hv1, ev1, ep1, gv,
             semA1, semD1, semH0, semH1, semE):
        srcn = (srcn0, srcn1)
        dstn = (dstn0, dstn1)
        asv = (asv1, asv1)
        adv = (adv1, adv1)
        hv = (hv0, hv1)
        ev = (ev1, ev1)
        ep = (ep1, ep1)
        semA = (semA1, semA1)
        semD = (semD1, semD1)
        semH = (semH0, semH1)

        cid = lax.axis_index("c")
        sid = lax.axis_index("s")
        wid = cid * NS + sid
        rows0 = sid * RPS
        pltpu.sync_copy(z128_h.at[pl.ds(rows0, RPS)], msg_s.at[pl.ds(rows0, RPS)])
        pltpu.sync_copy(z16_h.at[pl.ds(rows0, RPS)], sum_s.at[pl.ds(rows0, RPS)])
        pltpu.sync_copy(g_h, gv)
        plsc.subcore_barrier()
        g = gv[...]
        mask8 = lax.iota(jnp.int32, 16) < 8

        def compute(p, b, cp_a, cp_d, cp_h, prefetch):
            base = wid * epw + b * BA
            if cp_a is None:
                pltpu.make_async_copy(as_h.at[srcn[p]], asv[p], semA[p]).wait()
                pltpu.make_async_copy(ad_h.at[dstn[p]], adv[p], semD[p]).wait()
            else:
                cp_a.wait()
                cp_d.wait()

            def exp_i(j, _):
                t = asv[p][j, :] + adv[p][j, :]
                t = jnp.maximum(t, t * jnp.float32(0.2))
                ev[p][j, :] = jnp.exp(t - g)
                return 0

            lax.fori_loop(0, BA, exp_i, 0, unroll=4)

            def pack_i(j, _):
                e0 = ev[p][2 * j, :]
                e1 = ev[p][2 * j + 1, :]
                ep[p][j, :] = jnp.where(mask8, e0, e1)
                return 0

            lax.fori_loop(0, BA // 2, pack_i, 0, unroll=4)
            if cp_h is None:
                pltpu.make_async_copy(h_h.at[srcn[p]], hv[p], semH[p]).wait()
            else:
                cp_h.wait()

            cps = prefetch()

            def msg_i(e, _):
                erow = ev[p][e, :]
                for hh in range(H):
                    coef = _bcast_lane(erow, hh)
                    hv[p][e, pl.ds(hh * C, C)] = (
                        hv[p][e, pl.ds(hh * C, C)] * coef)
                return 0

            lax.fori_loop(0, BA, msg_i, 0, unroll=2)
            w_ep = pltpu.async_copy(
                ep[p], exp_h.at[pl.ds(base // 2, BA // 2)], semE)
            pltpu.sync_copy(ev[p], sum_s.at[dstn[p]], add=True)
            pltpu.sync_copy(hv[p], msg_s.at[dstn[p]], add=True)
            w_ep.wait()
            return cps

        def issue(p, b):
            base = wid * epw + b * BA
            pltpu.sync_copy(src_h.at[pl.ds(base, BA)], srcn[p])
            pltpu.sync_copy(dst_h.at[pl.ds(base, BA)], dstn[p])
            return (pltpu.async_copy(as_h.at[srcn[p]], asv[p], semA[p]),
                    pltpu.async_copy(ad_h.at[dstn[p]], adv[p], semD[p]),
                    pltpu.async_copy(h_h.at[srcn[p]], hv[p], semH[p]))

        issue(0, 0)

        def pair(g2, _):
            b0 = g2 * 2
            cps1 = compute(0, b0, None, None, None,
                           lambda: issue(1, b0 + 1))

            def pf2():
                @pl.when(b0 + 2 < nblk)
                def _():
                    issue(0, b0 + 2)
                return None

            compute(1, b0 + 1, cps1[0], cps1[1], cps1[2], pf2)
            return 0

        lax.fori_loop(0, nblk // 2, pair, 0)
        plsc.subcore_barrier()
        pltpu.sync_copy(msg_s.at[pl.ds(rows0, RPS)], macc_h.at[cid, pl.ds(rows0, RPS)])
        pltpu.sync_copy(sum_s.at[pl.ds(rows0, RPS)], sacc_h.at[cid, pl.ds(rows0, RPS)])

    f = pl.kernel(
        body,
        out_type=(
            jax.ShapeDtypeStruct((epad // 2, 2 * H), jnp.float32),
            jax.ShapeDtypeStruct((NC, NPAD, D), jnp.float32),
            jax.ShapeDtypeStruct((NC, NPAD, 2 * H), jnp.float32),
        ),
        mesh=mesh,
        compiler_params=pltpu.CompilerParams(use_tc_tiling_on_sc=False),
        scratch_types=(
            [pltpu.VMEM_SHARED((NPAD, D), jnp.float32),
             pltpu.VMEM_SHARED((NPAD, 2 * H), jnp.float32)]
            + [pltpu.VMEM((BA,), jnp.int32)] * 4
            + [pltpu.VMEM((BA, 2 * H), jnp.float32)] * 2
            + [pltpu.VMEM((BA, D), jnp.float32)] * 2
            + [pltpu.VMEM((BA, 2 * H), jnp.float32)] * 1
            + [pltpu.VMEM((BA // 2, 2 * H), jnp.float32)] * 1
            + [pltpu.VMEM((16,), jnp.float32)]
            + [pltpu.SemaphoreType.DMA] * 5
        ),
    )
    return f(src, dst, atab_s, atab_d, htab, g16, z128, z16)


def _finalize(macc, sacc, bias, dst, exp16, epad):
    epw = epad // NW
    nblk = epw // B
    assert nblk % 2 == 0
    mesh = plsc.VectorSubcoreMesh(core_axis_name="c", subcore_axis_name="s",
                                  num_cores=NC, num_subcores=NS)
    RHALF = NPAD // NC
    RQ = RPW // 2

    def body(macc_h, sacc_h, bias_h, dst_h, exp_h,
             out_h, alpha_h,
             inv_s, s0, s1, invv, inv157, m0, m1, biasv,
             dstn0, dstn1, ivv0, ivv1, epk0, epk1, av0, av1,
             semM0, semM1, semI0, semI1, semE0, semE1, semW0, semW1):
        dstn = (dstn0, dstn1)
        ivv = (ivv0, ivv1)
        epk = (epk0, epk1)
        av = (av0, av1)
        semI = (semI0, semI1)
        semE = (semE0, semE1)
        semW = (semW0, semW1)

        cid = lax.axis_index("c")
        sid = lax.axis_index("s")
        wid = cid * NS + sid
        mask8 = lax.iota(jnp.int32, 16) < 8

        r6 = sid * RPS
        cp0 = pltpu.async_copy(sacc_h.at[0, pl.ds(r6, RPS)], s0, semM0)
        cp1 = pltpu.async_copy(sacc_h.at[1, pl.ds(r6, RPS)], s1, semM1)
        pltpu.sync_copy(bias_h, biasv)
        cp0.wait()
        cp1.wait()

        def inv_i(j, _):
            s = s0[j, :] + s1[j, :]
            invv[j, :] = jnp.float32(1.0) / (s + jnp.float32(1e-16))
            return 0

        lax.fori_loop(0, RPS, inv_i, 0, unroll=4)
        pltpu.sync_copy(invv, inv_s.at[pl.ds(r6, RPS)])
        plsc.subcore_barrier()

        for k in range(2):
            r0 = cid * RHALF + sid * RPW + k * RQ
            cpm0 = pltpu.async_copy(macc_h.at[0, pl.ds(r0, RQ)], m0, semM0)
            cpm1 = pltpu.async_copy(macc_h.at[1, pl.ds(r0, RQ)], m1, semM1)
            pltpu.sync_copy(inv_s.at[pl.ds(r0, RQ)], inv157)
            cpm0.wait()
            cpm1.wait()

            def out_i(e, _):
                irow = inv157[e, :]
                for hh in range(H):
                    iv = _bcast_lane(irow, hh)
                    m = m0[e, pl.ds(hh * C, C)] + m1[e, pl.ds(hh * C, C)]
                    m0[e, pl.ds(hh * C, C)] = m * iv + biasv[pl.ds(hh * C, C)]
                return 0

            lax.fori_loop(0, RQ, out_i, 0)
            pltpu.sync_copy(m0, out_h.at[pl.ds(r0, RQ)])

        def issue(p, b):
            base = wid * epw + b * B
            pltpu.sync_copy(dst_h.at[pl.ds(base, B)], dstn[p])
            return (pltpu.async_copy(inv_s.at[dstn[p]], ivv[p], semI[p]),
                    pltpu.async_copy(
                        exp_h.at[pl.ds(base // 2, B // 2)], epk[p], semE[p]))

        def compute(p, b, cps, prefetch):
            base = wid * epw + b * B
            if cps is None:
                pltpu.make_async_copy(inv_s.at[dstn[p]], ivv[p], semI[p]).wait()
                pltpu.make_async_copy(
                    exp_h.at[pl.ds(base // 2, B // 2)], epk[p], semE[p]).wait()
            else:
                cps[0].wait()
                cps[1].wait()

            out = prefetch()

            def mul_i(j, _):
                iv = jnp.where(mask8, ivv[p][2 * j, :], ivv[p][2 * j + 1, :])
                av[p][j, :] = epk[p][j, :] * iv
                return 0

            lax.fori_loop(0, B // 2, mul_i, 0, unroll=4)
            w = pltpu.async_copy(
                av[p], alpha_h.at[pl.ds(base // 2, B // 2)], semW[p])
            return (out, w)

        issue(0, 0)

        def pair(g2, _):
            b0 = g2 * 2
            cps1, w0 = compute(0, b0, None, lambda: issue(1, b0 + 1))

            def pf2():
                @pl.when(b0 + 2 < nblk)
                def _():
                    issue(0, b0 + 2)
                return None

            _, w1 = compute(1, b0 + 1, cps1, pf2)
            w0.wait()
            w1.wait()
            return 0

        lax.fori_loop(0, nblk // 2, pair, 0)

    f = pl.kernel(
        body,
        out_type=(
            jax.ShapeDtypeStruct((NPAD, D), jnp.float32),
            jax.ShapeDtypeStruct((epad // 2, 2 * H), jnp.float32),
        ),
        mesh=mesh,
        compiler_params=pltpu.CompilerParams(use_tc_tiling_on_sc=False),
        scratch_types=(
            [pltpu.VMEM_SHARED((NPAD, 2 * H), jnp.float32)]
            + [pltpu.VMEM((RPS, 2 * H), jnp.float32)] * 3
            + [pltpu.VMEM((RPW // 2, 2 * H), jnp.float32)]
            + [pltpu.VMEM((RPW // 2, D), jnp.float32)] * 2
            + [pltpu.VMEM((D,), jnp.float32)]
            + [pltpu.VMEM((B,), jnp.int32)] * 2
            + [pltpu.VMEM((B, 2 * H), jnp.float32)] * 2
            + [pltpu.VMEM((B // 2, 2 * H), jnp.float32)] * 2
            + [pltpu.VMEM((B // 2, 2 * H), jnp.float32)] * 2
            + [pltpu.SemaphoreType.DMA] * 8
        ),
    )
    return f(macc, sacc, bias, dst, exp16)


def kernel(x, edge_index, W, att_src, att_dst, bias):
    n = x.shape[0]
    e = edge_index.shape[1]
    ne = e + n
    epad = ((ne + 2 * NW * B - 1) // (2 * NW * B)) * (2 * NW * B)

    loop = jnp.arange(n, dtype=edge_index.dtype)
    ei = jnp.concatenate([edge_index, jnp.stack([loop, loop], axis=0)], axis=1)
    padi = jnp.full((epad - ne,), n, jnp.int32)
    src = jnp.concatenate([ei[0], padi])
    dst = jnp.concatenate([ei[1], padi])

    xp = jnp.pad(x, ((0, NPAD - n), (0, 0)))
    hsel = jnp.repeat(jnp.arange(H), C)
    eye = jax.nn.one_hot(hsel, H, dtype=jnp.float32)
    s_src1 = eye * att_src.reshape(-1)[:, None]
    s_dst1 = eye * att_dst.reshape(-1)[:, None]
    s_src = jnp.concatenate([s_src1, s_src1], axis=1)
    s_dst = jnp.concatenate([s_dst1, s_dst1], axis=1)

    htab, atab_s, atab_d, mxs, mxd = _tc_proj(xp, W.T, s_src, s_dst)

    t = mxs[0] + mxd[0]
    g16 = jnp.where(t > 0, t, 0.2 * t)

    z128 = jnp.zeros((NPAD, D), jnp.float32)
    z16 = jnp.zeros((NPAD, 2 * H), jnp.float32)

    exp16, macc, sacc = _edge_accum(src, dst, atab_s, atab_d, htab,
                                    g16, z128, z16, epad)
    out_full, alpha_packed = _finalize(macc, sacc, bias, dst, exp16, epad)

    out = out_full[:n]
    alpha = alpha_packed.reshape(epad, H)[:ne]
    return out, ei, alpha


# speedup vs baseline: 1.1024x; 1.1024x over previous
import jax
import jax.numpy as jnp
from jax import lax
from jax.experimental import pallas as pl
from jax.experimental.pallas import tpu as pltpu
from jax.experimental.pallas import tpu_sc as plsc

N = 10000
D = 128
H = 8
C = 16
NC = 2
NS = 16
NW = NC * NS
BA = 112
B = 112

NPAD = 10240
RPW = NPAD // NW
RPS = NPAD // NS


def _bcast_lane(row16, lane):
    idx = jnp.full((16, 1), lane, jnp.int32)
    dnums = lax.GatherDimensionNumbers(
        offset_dims=(), collapsed_slice_dims=(0,), start_index_map=(0,))
    return lax.gather(row16, idx, dnums, (1,),
                      mode=lax.GatherScatterMode.PROMISE_IN_BOUNDS)


def _tc_proj_body(x_ref, wt_ref, ss_ref, sd_ref,
                  h_ref, as_ref, ad_ref, mxs_ref, mxd_ref):
    i = pl.program_id(0)
    h = jnp.dot(x_ref[...], wt_ref[...], preferred_element_type=jnp.float32)
    h_ref[...] = h
    a_s = jnp.dot(h, ss_ref[...], preferred_element_type=jnp.float32)
    a_d = jnp.dot(h, sd_ref[...], preferred_element_type=jnp.float32)
    as_ref[...] = a_s
    ad_ref[...] = a_d
    ms = jnp.broadcast_to(jnp.max(a_s, axis=0, keepdims=True), (8, 2 * H))
    md = jnp.broadcast_to(jnp.max(a_d, axis=0, keepdims=True), (8, 2 * H))

    @pl.when(i == 0)
    def _():
        mxs_ref[...] = ms
        mxd_ref[...] = md

    @pl.when(i > 0)
    def _():
        mxs_ref[...] = jnp.maximum(mxs_ref[...], ms)
        mxd_ref[...] = jnp.maximum(mxd_ref[...], md)


def _tc_proj(xp, wt, s_src, s_dst):
    rb = 256
    assert NPAD % rb == 0
    grid = (NPAD // rb,)
    return pl.pallas_call(
        _tc_proj_body,
        grid=grid,
        in_specs=[
            pl.BlockSpec((rb, D), lambda i: (i, 0)),
            pl.BlockSpec((D, D), lambda i: (0, 0)),
            pl.BlockSpec((D, 2 * H), lambda i: (0, 0)),
            pl.BlockSpec((D, 2 * H), lambda i: (0, 0)),
        ],
        out_specs=[
            pl.BlockSpec((rb, D), lambda i: (i, 0)),
            pl.BlockSpec((rb, 2 * H), lambda i: (i, 0)),
            pl.BlockSpec((rb, 2 * H), lambda i: (i, 0)),
            pl.BlockSpec((8, 2 * H), lambda i: (0, 0)),
            pl.BlockSpec((8, 2 * H), lambda i: (0, 0)),
        ],
        out_shape=[
            jax.ShapeDtypeStruct((NPAD, D), jnp.float32),
            jax.ShapeDtypeStruct((NPAD, 2 * H), jnp.float32),
            jax.ShapeDtypeStruct((NPAD, 2 * H), jnp.float32),
            jax.ShapeDtypeStruct((8, 2 * H), jnp.float32),
            jax.ShapeDtypeStruct((8, 2 * H), jnp.float32),
        ],
    )(xp, wt, s_src, s_dst)


def _edge_accum(src, dst, atab_s, atab_d, htab, g16, z128, z16, epad):
    epw = epad // NW
    nblk = epw // BA
    assert nblk % 2 == 0
    mesh = plsc.VectorSubcoreMesh(core_axis_name="c", subcore_axis_name="s",
                                  num_cores=NC, num_subcores=NS)

    def body(src_h, dst_h, as_h, ad_h, h_h, g_h, z128_h, z16_h,
             exp_h, macc_h, sacc_h,
             msg_s, sum_s,
             srcn0, srcn1, dstn0, dstn1,
             asv1, adv1, hv0, hv1, ev1, ep1, gv,
             semA1, semD1, semH0, semH1, semE):
        srcn = (srcn0, srcn1)
        dstn = (dstn0, dstn1)
        asv = (asv1, asv1)
        adv = (adv1, adv1)
        hv = (hv0, hv1)
        ev = (ev1, ev1)
        ep = (ep1, ep1)
        semA = (semA1, semA1)
        semD = (semD1, semD1)
        semH = (semH0, semH1)

        cid = lax.axis_index("c")
        sid = lax.axis_index("s")
        wid = cid * NS + sid
        rows0 = sid * RPS
        pltpu.sync_copy(z128_h.at[pl.ds(rows0, RPS)], msg_s.at[pl.ds(rows0, RPS)])
        pltpu.sync_copy(z16_h.at[pl.ds(rows0, RPS)], sum_s.at[pl.ds(rows0, RPS)])
        pltpu.sync_copy(g_h, gv)
        plsc.subcore_barrier()
        g = gv[...]
        mask8 = lax.iota(jnp.int32, 16) < 8

        def compute(p, b, cp_a, cp_d, cp_h, prefetch):
            base = wid * epw + b * BA
            if cp_a is None:
                pltpu.make_async_copy(as_h.at[srcn[p]], asv[p], semA[p]).wait()
                pltpu.make_async_copy(ad_h.at[dstn[p]], adv[p], semD[p]).wait()
            else:
                cp_a.wait()
                cp_d.wait()

            def exp_i(j, _):
                t = asv[p][j, :] + adv[p][j, :]
                t = jnp.maximum(t, t * jnp.float32(0.2))
                ev[p][j, :] = jnp.exp(t - g)
                return 0

            lax.fori_loop(0, BA, exp_i, 0, unroll=4)

            def pack_i(j, _):
                e0 = ev[p][2 * j, :]
                e1 = ev[p][2 * j + 1, :]
                ep[p][j, :] = jnp.where(mask8, e0, e1)
                return 0

            lax.fori_loop(0, BA // 2, pack_i, 0, unroll=4)
            if cp_h is None:
                pltpu.make_async_copy(h_h.at[srcn[p]], hv[p], semH[p]).wait()
            else:
                cp_h.wait()

            cps = prefetch()

            def msg_i(e, _):
                erow = ev[p][e, :]
                for hh in range(H):
                    coef = _bcast_lane(erow, hh)
                    hv[p][e, pl.ds(hh * C, C)] = (
                        hv[p][e, pl.ds(hh * C, C)] * coef)
                return 0

            lax.fori_loop(0, BA, msg_i, 0, unroll=2)
            w_ep = pltpu.async_copy(
                ep[p], exp_h.at[pl.ds(base // 2, BA // 2)], semE)
            pltpu.sync_copy(ev[p], sum_s.at[dstn[p]], add=True)
            pltpu.sync_copy(hv[p], msg_s.at[dstn[p]], add=True)
            w_ep.wait()
            return cps

        def issue(p, b):
            base = wid * epw + b * BA
            pltpu.sync_copy(src_h.at[pl.ds(base, BA)], srcn[p])
            pltpu.sync_copy(dst_h.at[pl.ds(base, BA)], dstn[p])
            return (pltpu.async_copy(as_h.at[srcn[p]], asv[p], semA[p]),
                    pltpu.async_copy(ad_h.at[dstn[p]], adv[p], semD[p]),
                    pltpu.async_copy(h_h.at[srcn[p]], hv[p], semH[p]))

        issue(0, 0)

        def pair(g2, _):
            b0 = g2 * 2
            cps1 = compute(0, b0, None, None, None,
                           lambda: issue(1, b0 + 1))

            def pf2():
                @pl.when(b0 + 2 < nblk)
                def _():
                    issue(0, b0 + 2)
                return None

            compute(1, b0 + 1, cps1[0], cps1[1], cps1[2], pf2)
            return 0

        lax.fori_loop(0, nblk // 2, pair, 0)
        plsc.subcore_barrier()
        pltpu.sync_copy(msg_s.at[pl.ds(rows0, RPS)], macc_h.at[cid, pl.ds(rows0, RPS)])
        pltpu.sync_copy(sum_s.at[pl.ds(rows0, RPS)], sacc_h.at[cid, pl.ds(rows0, RPS)])

    f = pl.kernel(
        body,
        out_type=(
            jax.ShapeDtypeStruct((epad // 2, 2 * H), jnp.float32),
            jax.ShapeDtypeStruct((NC, NPAD, D), jnp.float32),
            jax.ShapeDtypeStruct((NC, NPAD, 2 * H), jnp.float32),
        ),
        mesh=mesh,
        compiler_params=pltpu.CompilerParams(use_tc_tiling_on_sc=False),
        scratch_types=(
            [pltpu.VMEM_SHARED((NPAD, D), jnp.float32),
             pltpu.VMEM_SHARED((NPAD, 2 * H), jnp.float32)]
            + [pltpu.VMEM((BA,), jnp.int32)] * 4
            + [pltpu.VMEM((BA, 2 * H), jnp.float32)] * 2
            + [pltpu.VMEM((BA, D), jnp.float32)] * 2
            + [pltpu.VMEM((BA, 2 * H), jnp.float32)] * 1
            + [pltpu.VMEM((BA // 2, 2 * H), jnp.float32)] * 1
            + [pltpu.VMEM((16,), jnp.float32)]
            + [pltpu.SemaphoreType.DMA] * 5
        ),
    )
    return f(src, dst, atab_s, atab_d, htab, g16, z128, z16)


def _finalize(macc, sacc, bias, dst, exp16, epad):
    epw = epad // NW
    nblk = epw // B
    assert nblk % 2 == 0
    mesh = plsc.VectorSubcoreMesh(core_axis_name="c", subcore_axis_name="s",
                                  num_cores=NC, num_subcores=NS)
    RHALF = NPAD // NC
    RQ = RPW // 2

    def body(macc_h, sacc_h, bias_h, dst_h, exp_h,
             out_h, alpha_h,
             inv_s, s0, s1, invv, inv157, m0, m1, biasv,
             dstn0, dstn1, ivv0, ivv1, epk0, epk1, av0, av1,
             semM0, semM1, semI0, semI1, semE0, semE1, semW0, semW1):
        dstn = (dstn0, dstn1)
        ivv = (ivv0, ivv1)
        epk = (epk0, epk1)
        av = (av0, av1)
        semI = (semI0, semI1)
        semE = (semE0, semE1)
        semW = (semW0, semW1)

        cid = lax.axis_index("c")
        sid = lax.axis_index("s")
        wid = cid * NS + sid
        mask8 = lax.iota(jnp.int32, 16) < 8

        r6 = sid * RPS
        cp0 = pltpu.async_copy(sacc_h.at[0, pl.ds(r6, RPS)], s0, semM0)
        cp1 = pltpu.async_copy(sacc_h.at[1, pl.ds(r6, RPS)], s1, semM1)
        pltpu.sync_copy(bias_h, biasv)
        cp0.wait()
        cp1.wait()

        def inv_i(j, _):
            s = s0[j, :] + s1[j, :]
            invv[j, :] = jnp.float32(1.0) / (s + jnp.float32(1e-16))
            return 0

        lax.fori_loop(0, RPS, inv_i, 0, unroll=4)
        pltpu.sync_copy(invv, inv_s.at[pl.ds(r6, RPS)])
        plsc.subcore_barrier()

        for k in range(2):
            r0 = cid * RHALF + sid * RPW + k * RQ
            cpm0 = pltpu.async_copy(macc_h.at[0, pl.ds(r0, RQ)], m0, semM0)
            cpm1 = pltpu.async_copy(macc_h.at[1, pl.ds(r0, RQ)], m1, semM1)
            pltpu.sync_copy(inv_s.at[pl.ds(r0, RQ)], inv157)
            cpm0.wait()
            cpm1.wait()

            def out_i(e, _):
                irow = inv157[e, :]
                for hh in range(H):
                    iv = _bcast_lane(irow, hh)
                    m = m0[e, pl.ds(hh * C, C)] + m1[e, pl.ds(hh * C, C)]
                    m0[e, pl.ds(hh * C, C)] = m * iv + biasv[pl.ds(hh * C, C)]
                return 0

            lax.fori_loop(0, RQ, out_i, 0)
            pltpu.sync_copy(m0, out_h.at[pl.ds(r0, RQ)])

        def issue(p, b):
            base = wid * epw + b * B
            pltpu.sync_copy(dst_h.at[pl.ds(base, B)], dstn[p])
            return (pltpu.async_copy(inv_s.at[dstn[p]], ivv[p], semI[p]),
                    pltpu.async_copy(
                        exp_h.at[pl.ds(base // 2, B // 2)], epk[p], semE[p]))

        def compute(p, b, cps, prefetch):
            base = wid * epw + b * B
            if cps is None:
                pltpu.make_async_copy(inv_s.at[dstn[p]], ivv[p], semI[p]).wait()
                pltpu.make_async_copy(
                    exp_h.at[pl.ds(base // 2, B // 2)], epk[p], semE[p]).wait()
            else:
                cps[0].wait()
                cps[1].wait()

            out = prefetch()

            def mul_i(j, _):
                iv = jnp.where(mask8, ivv[p][2 * j, :], ivv[p][2 * j + 1, :])
                av[p][j, :] = epk[p][j, :] * iv
                return 0

            lax.fori_loop(0, B // 2, mul_i, 0, unroll=4)
            w = pltpu.async_copy(
                av[p], alpha_h.at[pl.ds(base // 2, B // 2)], semW[p])
            return (out, w)

        issue(0, 0)

        def pair(g2, _):
            b0 = g2 * 2
            cps1, w0 = compute(0, b0, None, lambda: issue(1, b0 + 1))

            def pf2():
                @pl.when(b0 + 2 < nblk)
                def _():
                    issue(0, b0 + 2)
                return None

            _, w1 = compute(1, b0 + 1, cps1, pf2)
            w0.wait()
            w1.wait()
            return 0

        lax.fori_loop(0, nblk // 2, pair, 0)

    f = pl.kernel(
        body,
        out_type=(
            jax.ShapeDtypeStruct((NPAD, D), jnp.float32),
            jax.ShapeDtypeStruct((epad // 2, 2 * H), jnp.float32),
        ),
        mesh=mesh,
        compiler_params=pltpu.CompilerParams(use_tc_tiling_on_sc=False),
        scratch_types=(
            [pltpu.VMEM_SHARED((NPAD, 2 * H), jnp.float32)]
            + [pltpu.VMEM((RPS, 2 * H), jnp.float32)] * 3
            + [pltpu.VMEM((RPW // 2, 2 * H), jnp.float32)]
            + [pltpu.VMEM((RPW // 2, D), jnp.float32)] * 2
            + [pltpu.VMEM((D,), jnp.float32)]
            + [pltpu.VMEM((B,), jnp.int32)] * 2
            + [pltpu.VMEM((B, 2 * H), jnp.float32)] * 2
            + [pltpu.VMEM((B // 2, 2 * H), jnp.float32)] * 2
            + [pltpu.VMEM((B // 2, 2 * H), jnp.float32)] * 2
            + [pltpu.SemaphoreType.DMA] * 8
        ),
    )
    return f(macc, sacc, bias, dst, exp16)


def kernel(x, edge_index, W, att_src, att_dst, bias):
    n = x.shape[0]
    e = edge_index.shape[1]
    ne = e + n
    assert B == BA
    epad = ((ne + 2 * NW * B - 1) // (2 * NW * B)) * (2 * NW * B)

    loop = jnp.arange(n, dtype=edge_index.dtype)
    ei = jnp.concatenate([edge_index, jnp.stack([loop, loop], axis=0)], axis=1)
    padi = jnp.full((epad - ne,), n, jnp.int32)
    src = jnp.concatenate([ei[0], padi])
    dst = jnp.concatenate([ei[1], padi])

    xp = jnp.pad(x, ((0, NPAD - n), (0, 0)))
    hsel = jnp.repeat(jnp.arange(H), C)
    eye = jax.nn.one_hot(hsel, H, dtype=jnp.float32)
    s_src1 = eye * att_src.reshape(-1)[:, None]
    s_dst1 = eye * att_dst.reshape(-1)[:, None]
    s_src = jnp.concatenate([s_src1, s_src1], axis=1)
    s_dst = jnp.concatenate([s_dst1, s_dst1], axis=1)

    htab, atab_s, atab_d, mxs, mxd = _tc_proj(xp, W.T, s_src, s_dst)

    t = mxs[0] + mxd[0]
    g16 = jnp.where(t > 0, t, 0.2 * t)

    z128 = jnp.zeros((NPAD, D), jnp.float32)
    z16 = jnp.zeros((NPAD, 2 * H), jnp.float32)

    exp16, macc, sacc = _edge_accum(src, dst, atab_s, atab_d, htab,
                                    g16, z128, z16, epad)
    out_full, alpha_packed = _finalize(macc, sacc, bias, dst, exp16, epad)

    out = out_full[:n]
    alpha = alpha_packed.reshape(epad, H)[:ne]
    return out, ei, alpha


# speedup vs baseline: 1.1180x; 1.0141x over previous
import jax
import jax.numpy as jnp
from jax import lax
from jax.experimental import pallas as pl
from jax.experimental.pallas import tpu as pltpu
from jax.experimental.pallas import tpu_sc as plsc

N = 10000
D = 128
H = 8
C = 16
NC = 2
NS = 16
NW = NC * NS
BA = 112
B = 112

NPAD = 10240
RPW = NPAD // NW
RPS = NPAD // NS


def _bcast_lane(row16, lane):
    idx = jnp.full((16, 1), lane, jnp.int32)
    dnums = lax.GatherDimensionNumbers(
        offset_dims=(), collapsed_slice_dims=(0,), start_index_map=(0,))
    return lax.gather(row16, idx, dnums, (1,),
                      mode=lax.GatherScatterMode.PROMISE_IN_BOUNDS)


def _tc_proj_body(x_ref, wt_ref, ss_ref, sd_ref,
                  h_ref, as_ref, ad_ref, mxs_ref, mxd_ref):
    i = pl.program_id(0)
    h = jnp.dot(x_ref[...], wt_ref[...], preferred_element_type=jnp.float32)
    h_ref[...] = h
    a_s = jnp.dot(h, ss_ref[...], preferred_element_type=jnp.float32)
    a_d = jnp.dot(h, sd_ref[...], preferred_element_type=jnp.float32)
    as_ref[...] = a_s
    ad_ref[...] = a_d
    ms = jnp.broadcast_to(jnp.max(a_s, axis=0, keepdims=True), (8, 2 * H))
    md = jnp.broadcast_to(jnp.max(a_d, axis=0, keepdims=True), (8, 2 * H))

    @pl.when(i == 0)
    def _():
        mxs_ref[...] = ms
        mxd_ref[...] = md

    @pl.when(i > 0)
    def _():
        mxs_ref[...] = jnp.maximum(mxs_ref[...], ms)
        mxd_ref[...] = jnp.maximum(mxd_ref[...], md)


def _tc_proj(xp, wt, s_src, s_dst):
    rb = 256
    assert NPAD % rb == 0
    grid = (NPAD // rb,)
    return pl.pallas_call(
        _tc_proj_body,
        grid=grid,
        in_specs=[
            pl.BlockSpec((rb, D), lambda i: (i, 0)),
            pl.BlockSpec((D, D), lambda i: (0, 0)),
            pl.BlockSpec((D, 2 * H), lambda i: (0, 0)),
            pl.BlockSpec((D, 2 * H), lambda i: (0, 0)),
        ],
        out_specs=[
            pl.BlockSpec((rb, D), lambda i: (i, 0)),
            pl.BlockSpec((rb, 2 * H), lambda i: (i, 0)),
            pl.BlockSpec((rb, 2 * H), lambda i: (i, 0)),
            pl.BlockSpec((8, 2 * H), lambda i: (0, 0)),
            pl.BlockSpec((8, 2 * H), lambda i: (0, 0)),
        ],
        out_shape=[
            jax.ShapeDtypeStruct((NPAD, D), jnp.float32),
            jax.ShapeDtypeStruct((NPAD, 2 * H), jnp.float32),
            jax.ShapeDtypeStruct((NPAD, 2 * H), jnp.float32),
            jax.ShapeDtypeStruct((8, 2 * H), jnp.float32),
            jax.ShapeDtypeStruct((8, 2 * H), jnp.float32),
        ],
    )(xp, wt, s_src, s_dst)


def _edge_accum(src, dst, atab_s, atab_d, htab, g16, z128, z16, epad):
    epw = epad // NW
    nblk = epw // BA
    assert nblk % 2 == 0
    mesh = plsc.VectorSubcoreMesh(core_axis_name="c", subcore_axis_name="s",
                                  num_cores=NC, num_subcores=NS)

    def body(src_h, dst_h, as_h, ad_h, h_h, g_h, z128_h, z16_h,
             exp_h, macc_h, sacc_h,
             msg_s, sum_s,
             srcn0, srcn1, dstn0, dstn1,
             asv1, adv1, hv0, hv1, ev1, ep1, gv,
             semA1, semD1, semH0, semH1, semE):
        srcn = (srcn0, srcn1)
        dstn = (dstn0, dstn1)
        asv = (asv1, asv1)
        adv = (adv1, adv1)
        hv = (hv0, hv1)
        ev = (ev1, ev1)
        ep = (ep1, ep1)
        semA = (semA1, semA1)
        semD = (semD1, semD1)
        semH = (semH0, semH1)

        cid = lax.axis_index("c")
        sid = lax.axis_index("s")
        wid = cid * NS + sid
        rows0 = sid * RPS
        pltpu.sync_copy(z128_h.at[pl.ds(rows0, RPS)], msg_s.at[pl.ds(rows0, RPS)])
        pltpu.sync_copy(z16_h.at[pl.ds(rows0, RPS)], sum_s.at[pl.ds(rows0, RPS)])
        pltpu.sync_copy(g_h, gv)
        plsc.subcore_barrier()
        g = gv[...]
        mask8 = lax.iota(jnp.int32, 16) < 8

        def compute(p, b, cp_a, cp_d, cp_h, prefetch):
            base = wid * epw + b * BA
            if cp_a is None:
                pltpu.make_async_copy(as_h.at[srcn[p]], asv[p], semA[p]).wait()
                pltpu.make_async_copy(ad_h.at[dstn[p]], adv[p], semD[p]).wait()
            else:
                cp_a.wait()
                cp_d.wait()

            def exp_i(j, _):
                t = asv[p][j, :] + adv[p][j, :]
                t = jnp.maximum(t, t * jnp.float32(0.2))
                ev[p][j, :] = jnp.exp(t - g)
                return 0

            lax.fori_loop(0, BA, exp_i, 0, unroll=4)

            def pack_i(j, _):
                e0 = ev[p][2 * j, :]
                e1 = ev[p][2 * j + 1, :]
                ep[p][j, :] = jnp.where(mask8, e0, e1)
                return 0

            lax.fori_loop(0, BA // 2, pack_i, 0, unroll=4)
            if cp_h is None:
                pltpu.make_async_copy(h_h.at[srcn[p]], hv[p], semH[p]).wait()
            else:
                cp_h.wait()

            cps = prefetch()

            @plsc.parallel_loop(0, BA, unroll=2)
            def msg_i(e):
                erow = ev[p][e, :]
                for hh in range(H):
                    coef = _bcast_lane(erow, hh)
                    hv[p][e, pl.ds(hh * C, C)] = (
                        hv[p][e, pl.ds(hh * C, C)] * coef)
            w_ep = pltpu.async_copy(
                ep[p], exp_h.at[pl.ds(base // 2, BA // 2)], semE)
            pltpu.sync_copy(ev[p], sum_s.at[dstn[p]], add=True)
            pltpu.sync_copy(hv[p], msg_s.at[dstn[p]], add=True)
            w_ep.wait()
            return cps

        def issue(p, b):
            base = wid * epw + b * BA
            pltpu.sync_copy(src_h.at[pl.ds(base, BA)], srcn[p])
            pltpu.sync_copy(dst_h.at[pl.ds(base, BA)], dstn[p])
            return (pltpu.async_copy(as_h.at[srcn[p]], asv[p], semA[p]),
                    pltpu.async_copy(ad_h.at[dstn[p]], adv[p], semD[p]),
                    pltpu.async_copy(h_h.at[srcn[p]], hv[p], semH[p]))

        issue(0, 0)

        def pair(g2, _):
            b0 = g2 * 2
            cps1 = compute(0, b0, None, None, None,
                           lambda: issue(1, b0 + 1))

            def pf2():
                @pl.when(b0 + 2 < nblk)
                def _():
                    issue(0, b0 + 2)
                return None

            compute(1, b0 + 1, cps1[0], cps1[1], cps1[2], pf2)
            return 0

        lax.fori_loop(0, nblk // 2, pair, 0)
        plsc.subcore_barrier()
        pltpu.sync_copy(msg_s.at[pl.ds(rows0, RPS)], macc_h.at[cid, pl.ds(rows0, RPS)])
        pltpu.sync_copy(sum_s.at[pl.ds(rows0, RPS)], sacc_h.at[cid, pl.ds(rows0, RPS)])

    f = pl.kernel(
        body,
        out_type=(
            jax.ShapeDtypeStruct((epad // 2, 2 * H), jnp.float32),
            jax.ShapeDtypeStruct((NC, NPAD, D), jnp.float32),
            jax.ShapeDtypeStruct((NC, NPAD, 2 * H), jnp.float32),
        ),
        mesh=mesh,
        compiler_params=pltpu.CompilerParams(use_tc_tiling_on_sc=False),
        scratch_types=(
            [pltpu.VMEM_SHARED((NPAD, D), jnp.float32),
             pltpu.VMEM_SHARED((NPAD, 2 * H), jnp.float32)]
            + [pltpu.VMEM((BA,), jnp.int32)] * 4
            + [pltpu.VMEM((BA, 2 * H), jnp.float32)] * 2
            + [pltpu.VMEM((BA, D), jnp.float32)] * 2
            + [pltpu.VMEM((BA, 2 * H), jnp.float32)] * 1
            + [pltpu.VMEM((BA // 2, 2 * H), jnp.float32)] * 1
            + [pltpu.VMEM((16,), jnp.float32)]
            + [pltpu.SemaphoreType.DMA] * 5
        ),
    )
    return f(src, dst, atab_s, atab_d, htab, g16, z128, z16)


def _finalize(macc, sacc, bias, dst, exp16, epad):
    epw = epad // NW
    nblk = epw // B
    assert nblk % 2 == 0
    mesh = plsc.VectorSubcoreMesh(core_axis_name="c", subcore_axis_name="s",
                                  num_cores=NC, num_subcores=NS)
    RHALF = NPAD // NC
    RQ = RPW // 2

    def body(macc_h, sacc_h, bias_h, dst_h, exp_h,
             out_h, alpha_h,
             inv_s, s0, s1, invv, inv157, m0, m1, biasv,
             dstn0, dstn1, ivv0, ivv1, epk0, epk1, av0, av1,
             semM0, semM1, semI0, semI1, semE0, semE1, semW0, semW1):
        dstn = (dstn0, dstn1)
        ivv = (ivv0, ivv1)
        epk = (epk0, epk1)
        av = (av0, av1)
        semI = (semI0, semI1)
        semE = (semE0, semE1)
        semW = (semW0, semW1)

        cid = lax.axis_index("c")
        sid = lax.axis_index("s")
        wid = cid * NS + sid
        mask8 = lax.iota(jnp.int32, 16) < 8

        r6 = sid * RPS
        cp0 = pltpu.async_copy(sacc_h.at[0, pl.ds(r6, RPS)], s0, semM0)
        cp1 = pltpu.async_copy(sacc_h.at[1, pl.ds(r6, RPS)], s1, semM1)
        pltpu.sync_copy(bias_h, biasv)
        cp0.wait()
        cp1.wait()

        def inv_i(j, _):
            s = s0[j, :] + s1[j, :]
            invv[j, :] = jnp.float32(1.0) / (s + jnp.float32(1e-16))
            return 0

        lax.fori_loop(0, RPS, inv_i, 0, unroll=4)
        pltpu.sync_copy(invv, inv_s.at[pl.ds(r6, RPS)])
        plsc.subcore_barrier()

        for k in range(2):
            r0 = cid * RHALF + sid * RPW + k * RQ
            cpm0 = pltpu.async_copy(macc_h.at[0, pl.ds(r0, RQ)], m0, semM0)
            cpm1 = pltpu.async_copy(macc_h.at[1, pl.ds(r0, RQ)], m1, semM1)
            pltpu.sync_copy(inv_s.at[pl.ds(r0, RQ)], inv157)
            cpm0.wait()
            cpm1.wait()

            def out_i(e, _):
                irow = inv157[e, :]
                for hh in range(H):
                    iv = _bcast_lane(irow, hh)
                    m = m0[e, pl.ds(hh * C, C)] + m1[e, pl.ds(hh * C, C)]
                    m0[e, pl.ds(hh * C, C)] = m * iv + biasv[pl.ds(hh * C, C)]
                return 0

            lax.fori_loop(0, RQ, out_i, 0)
            pltpu.sync_copy(m0, out_h.at[pl.ds(r0, RQ)])

        def issue(p, b):
            base = wid * epw + b * B
            pltpu.sync_copy(dst_h.at[pl.ds(base, B)], dstn[p])
            return (pltpu.async_copy(inv_s.at[dstn[p]], ivv[p], semI[p]),
                    pltpu.async_copy(
                        exp_h.at[pl.ds(base // 2, B // 2)], epk[p], semE[p]))

        def compute(p, b, cps, prefetch):
            base = wid * epw + b * B
            if cps is None:
                pltpu.make_async_copy(inv_s.at[dstn[p]], ivv[p], semI[p]).wait()
                pltpu.make_async_copy(
                    exp_h.at[pl.ds(base // 2, B // 2)], epk[p], semE[p]).wait()
            else:
                cps[0].wait()
                cps[1].wait()

            out = prefetch()

            def mul_i(j, _):
                iv = jnp.where(mask8, ivv[p][2 * j, :], ivv[p][2 * j + 1, :])
                av[p][j, :] = epk[p][j, :] * iv
                return 0

            lax.fori_loop(0, B // 2, mul_i, 0, unroll=4)
            w = pltpu.async_copy(
                av[p], alpha_h.at[pl.ds(base // 2, B // 2)], semW[p])
            return (out, w)

        issue(0, 0)

        def pair(g2, _):
            b0 = g2 * 2
            cps1, w0 = compute(0, b0, None, lambda: issue(1, b0 + 1))

            def pf2():
                @pl.when(b0 + 2 < nblk)
                def _():
                    issue(0, b0 + 2)
                return None

            _, w1 = compute(1, b0 + 1, cps1, pf2)
            w0.wait()
            w1.wait()
            return 0

        lax.fori_loop(0, nblk // 2, pair, 0)

    f = pl.kernel(
        body,
        out_type=(
            jax.ShapeDtypeStruct((NPAD, D), jnp.float32),
            jax.ShapeDtypeStruct((epad // 2, 2 * H), jnp.float32),
        ),
        mesh=mesh,
        compiler_params=pltpu.CompilerParams(use_tc_tiling_on_sc=False),
        scratch_types=(
            [pltpu.VMEM_SHARED((NPAD, 2 * H), jnp.float32)]
            + [pltpu.VMEM((RPS, 2 * H), jnp.float32)] * 3
            + [pltpu.VMEM((RPW // 2, 2 * H), jnp.float32)]
            + [pltpu.VMEM((RPW // 2, D), jnp.float32)] * 2
            + [pltpu.VMEM((D,), jnp.float32)]
            + [pltpu.VMEM((B,), jnp.int32)] * 2
            + [pltpu.VMEM((B, 2 * H), jnp.float32)] * 2
            + [pltpu.VMEM((B // 2, 2 * H), jnp.float32)] * 2
            + [pltpu.VMEM((B // 2, 2 * H), jnp.float32)] * 2
            + [pltpu.SemaphoreType.DMA] * 8
        ),
    )
    return f(macc, sacc, bias, dst, exp16)


def kernel(x, edge_index, W, att_src, att_dst, bias):
    n = x.shape[0]
    e = edge_index.shape[1]
    ne = e + n
    assert B == BA
    epad = ((ne + 2 * NW * B - 1) // (2 * NW * B)) * (2 * NW * B)

    loop = jnp.arange(n, dtype=edge_index.dtype)
    ei = jnp.concatenate([edge_index, jnp.stack([loop, loop], axis=0)], axis=1)
    padi = jnp.full((epad - ne,), n, jnp.int32)
    src = jnp.concatenate([ei[0], padi])
    dst = jnp.concatenate([ei[1], padi])

    xp = jnp.pad(x, ((0, NPAD - n), (0, 0)))
    hsel = jnp.repeat(jnp.arange(H), C)
    eye = jax.nn.one_hot(hsel, H, dtype=jnp.float32)
    s_src1 = eye * att_src.reshape(-1)[:, None]
    s_dst1 = eye * att_dst.reshape(-1)[:, None]
    s_src = jnp.concatenate([s_src1, s_src1], axis=1)
    s_dst = jnp.concatenate([s_dst1, s_dst1], axis=1)

    htab, atab_s, atab_d, mxs, mxd = _tc_proj(xp, W.T, s_src, s_dst)

    t = mxs[0] + mxd[0]
    g16 = jnp.where(t > 0, t, 0.2 * t)

    z128 = jnp.zeros((NPAD, D), jnp.float32)
    z16 = jnp.zeros((NPAD, 2 * H), jnp.float32)

    exp16, macc, sacc = _edge_accum(src, dst, atab_s, atab_d, htab,
                                    g16, z128, z16, epad)
    out_full, alpha_packed = _finalize(macc, sacc, bias, dst, exp16, epad)

    out = out_full[:n]
    alpha = alpha_packed.reshape(epad, H)[:ne]
    return out, ei, alpha


# speedup vs baseline: 1.2067x; 1.0793x over previous
import jax
import jax.numpy as jnp
from jax import lax
from jax.experimental import pallas as pl
from jax.experimental.pallas import tpu as pltpu
from jax.experimental.pallas import tpu_sc as plsc

N = 10000
D = 128
H = 8
C = 16
NC = 2
NS = 16
NW = NC * NS
BA = 112
B = 112

NPAD = 10240
RPW = NPAD // NW
RPS = NPAD // NS


def _bcast_lane(row16, lane):
    idx = jnp.full((16, 1), lane, jnp.int32)
    dnums = lax.GatherDimensionNumbers(
        offset_dims=(), collapsed_slice_dims=(0,), start_index_map=(0,))
    return lax.gather(row16, idx, dnums, (1,),
                      mode=lax.GatherScatterMode.PROMISE_IN_BOUNDS)


def _tc_proj_body(x_ref, wt_ref, ss_ref, sd_ref,
                  h_ref, as_ref, ad_ref, mxs_ref, mxd_ref):
    i = pl.program_id(0)
    h = jnp.dot(x_ref[...], wt_ref[...], preferred_element_type=jnp.float32)
    h_ref[...] = h
    a_s = jnp.dot(h, ss_ref[...], preferred_element_type=jnp.float32)
    a_d = jnp.dot(h, sd_ref[...], preferred_element_type=jnp.float32)
    as_ref[...] = a_s
    ad_ref[...] = a_d
    ms = jnp.broadcast_to(jnp.max(a_s, axis=0, keepdims=True), (8, 2 * H))
    md = jnp.broadcast_to(jnp.max(a_d, axis=0, keepdims=True), (8, 2 * H))

    @pl.when(i == 0)
    def _():
        mxs_ref[...] = ms
        mxd_ref[...] = md

    @pl.when(i > 0)
    def _():
        mxs_ref[...] = jnp.maximum(mxs_ref[...], ms)
        mxd_ref[...] = jnp.maximum(mxd_ref[...], md)


def _tc_proj(xp, wt, s_src, s_dst):
    rb = 256
    assert NPAD % rb == 0
    grid = (NPAD // rb,)
    return pl.pallas_call(
        _tc_proj_body,
        grid=grid,
        in_specs=[
            pl.BlockSpec((rb, D), lambda i: (i, 0)),
            pl.BlockSpec((D, D), lambda i: (0, 0)),
            pl.BlockSpec((D, 2 * H), lambda i: (0, 0)),
            pl.BlockSpec((D, 2 * H), lambda i: (0, 0)),
        ],
        out_specs=[
            pl.BlockSpec((rb, D), lambda i: (i, 0)),
            pl.BlockSpec((rb, 2 * H), lambda i: (i, 0)),
            pl.BlockSpec((rb, 2 * H), lambda i: (i, 0)),
            pl.BlockSpec((8, 2 * H), lambda i: (0, 0)),
            pl.BlockSpec((8, 2 * H), lambda i: (0, 0)),
        ],
        out_shape=[
            jax.ShapeDtypeStruct((NPAD, D), jnp.float32),
            jax.ShapeDtypeStruct((NPAD, 2 * H), jnp.float32),
            jax.ShapeDtypeStruct((NPAD, 2 * H), jnp.float32),
            jax.ShapeDtypeStruct((8, 2 * H), jnp.float32),
            jax.ShapeDtypeStruct((8, 2 * H), jnp.float32),
        ],
    )(xp, wt, s_src, s_dst)


def _edge_accum(src, dst, atab_s, atab_d, htab, g16, z128, z16, epad):
    epw = epad // NW
    nblk = epw // BA
    assert nblk % 2 == 0
    mesh = plsc.VectorSubcoreMesh(core_axis_name="c", subcore_axis_name="s",
                                  num_cores=NC, num_subcores=NS)

    def body(src_h, dst_h, as_h, ad_h, h_h, g_h, z128_h, z16_h,
             exp_h, macc_h, sacc_h,
             msg_s, sum_s,
             srcn0, srcn1, dstn0, dstn1,
             asv1, adv1, hv0, hv1, ev1, ep1, gv,
             semA1, semD1, semH0, semH1, semE):
        srcn = (srcn0, srcn1)
        dstn = (dstn0, dstn1)
        asv = (asv1, asv1)
        adv = (adv1, adv1)
        hv = (hv0, hv1)
        ev = (ev1, ev1)
        ep = (ep1, ep1)
        semA = (semA1, semA1)
        semD = (semD1, semD1)
        semH = (semH0, semH1)

        cid = lax.axis_index("c")
        sid = lax.axis_index("s")
        wid = cid * NS + sid
        rows0 = sid * RPS
        pltpu.sync_copy(z128_h.at[pl.ds(rows0, RPS)], msg_s.at[pl.ds(rows0, RPS)])
        pltpu.sync_copy(z16_h.at[pl.ds(rows0, RPS)], sum_s.at[pl.ds(rows0, RPS)])
        pltpu.sync_copy(g_h, gv)
        plsc.subcore_barrier()
        g = gv[...]
        mask8 = lax.iota(jnp.int32, 16) < 8

        def compute(p, b, cp_a, cp_d, cp_h, prefetch):
            base = wid * epw + b * BA
            if cp_a is None:
                pltpu.make_async_copy(as_h.at[srcn[p]], asv[p], semA[p]).wait()
                pltpu.make_async_copy(ad_h.at[dstn[p]], adv[p], semD[p]).wait()
            else:
                cp_a.wait()
                cp_d.wait()

            @plsc.parallel_loop(0, BA, unroll=4)
            def exp_i(j):
                t = asv[p][j, :] + adv[p][j, :]
                t = jnp.maximum(t, t * jnp.float32(0.2))
                ev[p][j, :] = jnp.exp(t - g)

            @plsc.parallel_loop(0, BA // 2, unroll=4)
            def pack_i(j):
                e0 = ev[p][2 * j, :]
                e1 = ev[p][2 * j + 1, :]
                ep[p][j, :] = jnp.where(mask8, e0, e1)
            if cp_h is None:
                pltpu.make_async_copy(h_h.at[srcn[p]], hv[p], semH[p]).wait()
            else:
                cp_h.wait()

            cps = prefetch()

            @plsc.parallel_loop(0, BA, unroll=2)
            def msg_i(e):
                erow = ev[p][e, :]
                for hh in range(H):
                    coef = _bcast_lane(erow, hh)
                    hv[p][e, pl.ds(hh * C, C)] = (
                        hv[p][e, pl.ds(hh * C, C)] * coef)
            w_ep = pltpu.async_copy(
                ep[p], exp_h.at[pl.ds(base // 2, BA // 2)], semE)
            pltpu.sync_copy(ev[p], sum_s.at[dstn[p]], add=True)
            pltpu.sync_copy(hv[p], msg_s.at[dstn[p]], add=True)
            w_ep.wait()
            return cps

        def issue(p, b):
            base = wid * epw + b * BA
            pltpu.sync_copy(src_h.at[pl.ds(base, BA)], srcn[p])
            pltpu.sync_copy(dst_h.at[pl.ds(base, BA)], dstn[p])
            return (pltpu.async_copy(as_h.at[srcn[p]], asv[p], semA[p]),
                    pltpu.async_copy(ad_h.at[dstn[p]], adv[p], semD[p]),
                    pltpu.async_copy(h_h.at[srcn[p]], hv[p], semH[p]))

        issue(0, 0)

        def pair(g2, _):
            b0 = g2 * 2
            cps1 = compute(0, b0, None, None, None,
                           lambda: issue(1, b0 + 1))

            def pf2():
                @pl.when(b0 + 2 < nblk)
                def _():
                    issue(0, b0 + 2)
                return None

            compute(1, b0 + 1, cps1[0], cps1[1], cps1[2], pf2)
            return 0

        lax.fori_loop(0, nblk // 2, pair, 0)
        plsc.subcore_barrier()
        pltpu.sync_copy(msg_s.at[pl.ds(rows0, RPS)], macc_h.at[cid, pl.ds(rows0, RPS)])
        pltpu.sync_copy(sum_s.at[pl.ds(rows0, RPS)], sacc_h.at[cid, pl.ds(rows0, RPS)])

    f = pl.kernel(
        body,
        out_type=(
            jax.ShapeDtypeStruct((epad // 2, 2 * H), jnp.float32),
            jax.ShapeDtypeStruct((NC, NPAD, D), jnp.float32),
            jax.ShapeDtypeStruct((NC, NPAD, 2 * H), jnp.float32),
        ),
        mesh=mesh,
        compiler_params=pltpu.CompilerParams(use_tc_tiling_on_sc=False),
        scratch_types=(
            [pltpu.VMEM_SHARED((NPAD, D), jnp.float32),
             pltpu.VMEM_SHARED((NPAD, 2 * H), jnp.float32)]
            + [pltpu.VMEM((BA,), jnp.int32)] * 4
            + [pltpu.VMEM((BA, 2 * H), jnp.float32)] * 2
            + [pltpu.VMEM((BA, D), jnp.float32)] * 2
            + [pltpu.VMEM((BA, 2 * H), jnp.float32)] * 1
            + [pltpu.VMEM((BA // 2, 2 * H), jnp.float32)] * 1
            + [pltpu.VMEM((16,), jnp.float32)]
            + [pltpu.SemaphoreType.DMA] * 5
        ),
    )
    return f(src, dst, atab_s, atab_d, htab, g16, z128, z16)


def _finalize(macc, sacc, bias, dst, exp16, epad):
    epw = epad // NW
    nblk = epw // B
    assert nblk % 2 == 0
    mesh = plsc.VectorSubcoreMesh(core_axis_name="c", subcore_axis_name="s",
                                  num_cores=NC, num_subcores=NS)
    RHALF = NPAD // NC
    RQ = RPW // 2

    def body(macc_h, sacc_h, bias_h, dst_h, exp_h,
             out_h, alpha_h,
             inv_s, s0, s1, invv, inv157, m0, m1, biasv,
             dstn0, dstn1, ivv0, ivv1, epk0, epk1, av0, av1,
             semM0, semM1, semI0, semI1, semE0, semE1, semW0, semW1):
        dstn = (dstn0, dstn1)
        ivv = (ivv0, ivv1)
        epk = (epk0, epk1)
        av = (av0, av1)
        semI = (semI0, semI1)
        semE = (semE0, semE1)
        semW = (semW0, semW1)

        cid = lax.axis_index("c")
        sid = lax.axis_index("s")
        wid = cid * NS + sid
        mask8 = lax.iota(jnp.int32, 16) < 8

        r6 = sid * RPS
        cp0 = pltpu.async_copy(sacc_h.at[0, pl.ds(r6, RPS)], s0, semM0)
        cp1 = pltpu.async_copy(sacc_h.at[1, pl.ds(r6, RPS)], s1, semM1)
        pltpu.sync_copy(bias_h, biasv)
        cp0.wait()
        cp1.wait()

        @plsc.parallel_loop(0, RPS, unroll=4)
        def inv_i(j):
            s = s0[j, :] + s1[j, :]
            invv[j, :] = jnp.float32(1.0) / (s + jnp.float32(1e-16))
        pltpu.sync_copy(invv, inv_s.at[pl.ds(r6, RPS)])
        plsc.subcore_barrier()

        for k in range(2):
            r0 = cid * RHALF + sid * RPW + k * RQ
            cpm0 = pltpu.async_copy(macc_h.at[0, pl.ds(r0, RQ)], m0, semM0)
            cpm1 = pltpu.async_copy(macc_h.at[1, pl.ds(r0, RQ)], m1, semM1)
            pltpu.sync_copy(inv_s.at[pl.ds(r0, RQ)], inv157)
            cpm0.wait()
            cpm1.wait()

            @plsc.parallel_loop(0, RQ, unroll=2)
            def out_i(e):
                irow = inv157[e, :]
                for hh in range(H):
                    iv = _bcast_lane(irow, hh)
                    m = m0[e, pl.ds(hh * C, C)] + m1[e, pl.ds(hh * C, C)]
                    m0[e, pl.ds(hh * C, C)] = m * iv + biasv[pl.ds(hh * C, C)]
            pltpu.sync_copy(m0, out_h.at[pl.ds(r0, RQ)])

        def issue(p, b):
            base = wid * epw + b * B
            pltpu.sync_copy(dst_h.at[pl.ds(base, B)], dstn[p])
            return (pltpu.async_copy(inv_s.at[dstn[p]], ivv[p], semI[p]),
                    pltpu.async_copy(
                        exp_h.at[pl.ds(base // 2, B // 2)], epk[p], semE[p]))

        def compute(p, b, cps, prefetch):
            base = wid * epw + b * B
            if cps is None:
                pltpu.make_async_copy(inv_s.at[dstn[p]], ivv[p], semI[p]).wait()
                pltpu.make_async_copy(
                    exp_h.at[pl.ds(base // 2, B // 2)], epk[p], semE[p]).wait()
            else:
                cps[0].wait()
                cps[1].wait()

            out = prefetch()

            @plsc.parallel_loop(0, B // 2, unroll=4)
            def mul_i(j):
                iv = jnp.where(mask8, ivv[p][2 * j, :], ivv[p][2 * j + 1, :])
                av[p][j, :] = epk[p][j, :] * iv
            w = pltpu.async_copy(
                av[p], alpha_h.at[pl.ds(base // 2, B // 2)], semW[p])
            return (out, w)

        issue(0, 0)

        def pair(g2, _):
            b0 = g2 * 2
            cps1, w0 = compute(0, b0, None, lambda: issue(1, b0 + 1))

            def pf2():
                @pl.when(b0 + 2 < nblk)
                def _():
                    issue(0, b0 + 2)
                return None

            _, w1 = compute(1, b0 + 1, cps1, pf2)
            w0.wait()
            w1.wait()
            return 0

        lax.fori_loop(0, nblk // 2, pair, 0)

    f = pl.kernel(
        body,
        out_type=(
            jax.ShapeDtypeStruct((NPAD, D), jnp.float32),
            jax.ShapeDtypeStruct((epad // 2, 2 * H), jnp.float32),
        ),
        mesh=mesh,
        compiler_params=pltpu.CompilerParams(use_tc_tiling_on_sc=False),
        scratch_types=(
            [pltpu.VMEM_SHARED((NPAD, 2 * H), jnp.float32)]
            + [pltpu.VMEM((RPS, 2 * H), jnp.float32)] * 3
            + [pltpu.VMEM((RPW // 2, 2 * H), jnp.float32)]
            + [pltpu.VMEM((RPW // 2, D), jnp.float32)] * 2
            + [pltpu.VMEM((D,), jnp.float32)]
            + [pltpu.VMEM((B,), jnp.int32)] * 2
            + [pltpu.VMEM((B, 2 * H), jnp.float32)] * 2
            + [pltpu.VMEM((B // 2, 2 * H), jnp.float32)] * 2
            + [pltpu.VMEM((B // 2, 2 * H), jnp.float32)] * 2
            + [pltpu.SemaphoreType.DMA] * 8
        ),
    )
    return f(macc, sacc, bias, dst, exp16)


def kernel(x, edge_index, W, att_src, att_dst, bias):
    n = x.shape[0]
    e = edge_index.shape[1]
    ne = e + n
    assert B == BA
    epad = ((ne + 2 * NW * B - 1) // (2 * NW * B)) * (2 * NW * B)

    loop = jnp.arange(n, dtype=edge_index.dtype)
    ei = jnp.concatenate([edge_index, jnp.stack([loop, loop], axis=0)], axis=1)
    padi = jnp.full((epad - ne,), n, jnp.int32)
    src = jnp.concatenate([ei[0], padi])
    dst = jnp.concatenate([ei[1], padi])

    xp = jnp.pad(x, ((0, NPAD - n), (0, 0)))
    hsel = jnp.repeat(jnp.arange(H), C)
    eye = jax.nn.one_hot(hsel, H, dtype=jnp.float32)
    s_src1 = eye * att_src.reshape(-1)[:, None]
    s_dst1 = eye * att_dst.reshape(-1)[:, None]
    s_src = jnp.concatenate([s_src1, s_src1], axis=1)
    s_dst = jnp.concatenate([s_dst1, s_dst1], axis=1)

    htab, atab_s, atab_d, mxs, mxd = _tc_proj(xp, W.T, s_src, s_dst)

    t = mxs[0] + mxd[0]
    g16 = jnp.where(t > 0, t, 0.2 * t)

    z128 = jnp.zeros((NPAD, D), jnp.float32)
    z16 = jnp.zeros((NPAD, 2 * H), jnp.float32)

    exp16, macc, sacc = _edge_accum(src, dst, atab_s, atab_d, htab,
                                    g16, z128, z16, epad)
    out_full, alpha_packed = _finalize(macc, sacc, bias, dst, exp16, epad)

    out = out_full[:n]
    alpha = alpha_packed.reshape(epad, H)[:ne]
    return out, ei, alpha
